# baseline (speedup 1.0000x reference)
import functools

import jax
import jax.numpy as jnp
from jax import lax
from jax.experimental import pallas as pl
from jax.experimental.pallas import tpu as pltpu
from jax.experimental.pallas import tpu_sc as plsc

N_NODES = 100000
N_EDGES = 6400000
SHORT_RANGE_CUTOFF = 0.2
LONG_RANGE_CUTOFF = 0.8
INV_LR2 = 1.0 / (LONG_RANGE_CUTOFF * LONG_RANGE_CUTOFF)
TWO_OVER_LR = 2.0 / LONG_RANGE_CUTOFF
KEHALF = 7.199822675975274

ROWS = 16
ROW_W = 128
BLK = ROWS * ROW_W
NBLK = N_EDGES // BLK
NTILES = 32
BLK_PER_TILE = NBLK // NTILES
BLK_REM = NBLK - BLK_PER_TILE * NTILES
WIN = BLK
ACC_PAD = 100224
ACC_SLICE = ACC_PAD // 16

_MESH = plsc.VectorSubcoreMesh(
    core_axis_name="c", subcore_axis_name="s", num_cores=2, num_subcores=16
)


def _rsqrt_f32(s):
    xi = plsc.bitcast(s, jnp.int32)
    yi = jnp.int32(0x5F3759DF) - lax.shift_right_arithmetic(xi, 1)
    y = plsc.bitcast(yi, jnp.float32)
    half_s = 0.5 * s
    for _ in range(3):
        y = y * (1.5 - half_s * y * y)
    return y


def _edge_energy(d, qi, qj):
    s = d * d + 1.0
    rs = _rsqrt_f32(s)
    ds = s * rs
    e_shl = rs + ds * INV_LR2 - TWO_OVER_LR
    diff = (1.0 / d - rs) + (d - ds) * INV_LR2
    x = jnp.minimum(d * (1.0 / SHORT_RANGE_CUTOFF), 1.0)
    sw = x * x * x * (x * (x * 6.0 - 15.0) + 10.0)
    e = (KEHALF * qi * qj) * (e_shl + sw * diff)
    return jnp.where(d <= LONG_RANGE_CUTOFF, e, 0.0)


@functools.partial(
    pl.kernel,
    out_type=jax.ShapeDtypeStruct((2 * ACC_PAD,), jnp.float32),
    mesh=_MESH,
    compiler_params=pltpu.CompilerParams(needs_layout_passes=False),
    scratch_types=[
        pltpu.VMEM((N_NODES,), jnp.float32),
        pltpu.VMEM((ROWS, ROW_W), jnp.int32),
        pltpu.VMEM((ROWS, ROW_W), jnp.int32),
        pltpu.VMEM((ROWS, ROW_W), jnp.float32),
        pltpu.VMEM((ROWS, ROW_W), jnp.int32),
        pltpu.VMEM((ROWS, ROW_W), jnp.int32),
        pltpu.VMEM((ROWS, ROW_W), jnp.float32),
        pltpu.VMEM((WIN // ROW_W, ROW_W), jnp.float32),
        pltpu.VMEM((WIN // ROW_W, ROW_W), jnp.int32),
        pltpu.VMEM((ACC_SLICE,), jnp.float32),
        pltpu.VMEM_SHARED((ACC_PAD,), jnp.float32),
        pltpu.SemaphoreType.DMA,
        pltpu.SemaphoreType.DMA,
        pltpu.SemaphoreType.DMA,
    ],
)
def _sc_energy(charges_hbm, dist_hbm, idxi_hbm, idxj_hbm, zeros_hbm, out_hbm,
               table_v, iiA, ijA, diA, iiB, ijB, diB, win_v, fidx_v, stage_v,
               acc_sh, semA_in, semB_in, sem_sc):
    c = lax.axis_index("c")
    s = lax.axis_index("s")
    wid = s * 2 + c
    iota16 = lax.iota(jnp.int32, 16)

    setA = (iiA, ijA, diA, semA_in)
    setB = (iiB, ijB, diB, semB_in)

    def fire_in(bufs, bg):
        ii, ij, di, sem = bufs
        return (
            pltpu.async_copy(idxi_hbm.at[bg], ii, sem),
            pltpu.async_copy(idxj_hbm.at[bg], ij, sem),
            pltpu.async_copy(dist_hbm.at[bg], di, sem),
        )

    def drain(descs):
        for desc in descs:
            desc.wait()

    def process_block(bufs):
        ii_v, ij_v, di_v, _ = bufs
        bmin = lax.reduce_min(ii_v[0, pl.ds(0, 16)], (0,))
        bmax = lax.reduce_max(ii_v[ROWS - 1, pl.ds(ROW_W - 16, 16)], (0,))
        span = bmax - bmin

        @pl.when(span < WIN)
        def _fast():
            def row_fold(r, _):
                for j in range(ROW_W // 16):
                    off = j * 16
                    ii = ii_v[r, pl.ds(off, 16)]
                    ij = ij_v[r, pl.ds(off, 16)]
                    d = di_v[r, pl.ds(off, 16)]
                    qi = plsc.load_gather(table_v, [ii])
                    qj = plsc.load_gather(table_v, [ij])
                    loc = ii - bmin
                    plsc.addupdate_scatter(
                        win_v,
                        [lax.shift_right_logical(loc, 7),
                         lax.bitwise_and(loc, ROW_W - 1)],
                        _edge_energy(d, qi, qj),
                    )
                return 0

            lax.fori_loop(0, ROWS, row_fold, 0)

            zeros16 = jnp.zeros((16,), jnp.float32)

            def flush_row(r, _):
                base = bmin + r * ROW_W
                for j in range(ROW_W // 16):
                    fidx_v[r, pl.ds(j * 16, 16)] = base + j * 16 + iota16
                pltpu.sync_copy(
                    win_v.at[r], acc_sh.at[fidx_v.at[r]], add=True
                )
                for j in range(ROW_W // 16):
                    win_v[r, pl.ds(j * 16, 16)] = zeros16
                return 0

            lax.fori_loop(0, lax.shift_right_logical(span, 7) + 1,
                          flush_row, 0)

        @pl.when(span >= WIN)
        def _wide():
            def row_body(r, _):
                for j in range(ROW_W // 16):
                    off = j * 16
                    ii = ii_v[r, pl.ds(off, 16)]
                    ij = ij_v[r, pl.ds(off, 16)]
                    d = di_v[r, pl.ds(off, 16)]
                    qi = plsc.load_gather(table_v, [ii])
                    qj = plsc.load_gather(table_v, [ij])
                    di_v[r, pl.ds(off, 16)] = _edge_energy(d, qi, qj)
                return 0

            lax.fori_loop(0, ROWS, row_body, 0)
            descs = [
                pltpu.async_copy(
                    di_v.at[r], acc_sh.at[ii_v.at[r]], sem_sc, add=True
                )
                for r in range(ROWS)
            ]
            for desc in descs:
                desc.wait()

    pltpu.sync_copy(charges_hbm, table_v)
    pltpu.sync_copy(zeros_hbm.at[pl.ds(s * ACC_SLICE, ACC_SLICE)], stage_v)
    pltpu.sync_copy(stage_v, acc_sh.at[pl.ds(s * ACC_SLICE, ACC_SLICE)])
    z16 = jnp.zeros((16,), jnp.float32)

    def zero_row(r, _):
        for j in range(ROW_W // 16):
            win_v[r, pl.ds(j * 16, 16)] = z16
        return 0

    lax.fori_loop(0, WIN // ROW_W, zero_row, 0)
    plsc.subcore_barrier()

    start_blk = wid * BLK_PER_TILE + jnp.minimum(wid, BLK_REM)

    def pair_body(p, _):
        inA = fire_in(setA, start_blk + 2 * p)
        inB = fire_in(setB, start_blk + 2 * p + 1)
        drain(inA)
        process_block(setA)
        drain(inB)
        process_block(setB)
        return 0

    lax.fori_loop(0, BLK_PER_TILE // 2, pair_body, 0)

    def tail_block(bg):
        inB = fire_in(setB, bg)
        drain(inB)
        process_block(setB)

    tail_block(start_blk + BLK_PER_TILE - 1)

    @pl.when(wid < BLK_REM)
    def _():
        tail_block(start_blk + BLK_PER_TILE)

    plsc.subcore_barrier()
    pltpu.sync_copy(acc_sh.at[pl.ds(s * ACC_SLICE, ACC_SLICE)], stage_v)
    pltpu.sync_copy(
        stage_v, out_hbm.at[pl.ds(c * ACC_PAD + s * ACC_SLICE, ACC_SLICE)]
    )


def _combine_body(p_ref, o_ref):
    o_ref[...] = p_ref[0] + p_ref[1]


def kernel(atomic_charges, distances, idx_i, idx_j):
    idx_i = idx_i.astype(jnp.int32).reshape(NBLK, ROWS, ROW_W)
    idx_j = idx_j.astype(jnp.int32).reshape(NBLK, ROWS, ROW_W)
    dist = distances.reshape(NBLK, ROWS, ROW_W)
    zeros = jnp.zeros((ACC_PAD,), jnp.float32)
    part = _sc_energy(atomic_charges, dist, idx_i, idx_j, zeros)
    part = part.reshape(2, ACC_PAD // 128, 128)
    summed = pl.pallas_call(
        _combine_body,
        out_shape=jax.ShapeDtypeStruct((ACC_PAD // 128, 128), jnp.float32),
    )(part)
    return summed.reshape(ACC_PAD)[:N_NODES]


# speedup vs baseline: 2.0773x; 2.0773x over previous
import functools

import jax
import jax.numpy as jnp
from jax import lax
from jax.experimental import pallas as pl
from jax.experimental.pallas import tpu as pltpu
from jax.experimental.pallas import tpu_sc as plsc

N_NODES = 100000
N_EDGES = 6400000
SHORT_RANGE_CUTOFF = 0.2
LONG_RANGE_CUTOFF = 0.8
INV_LR2 = 1.0 / (LONG_RANGE_CUTOFF * LONG_RANGE_CUTOFF)
TWO_OVER_LR = 2.0 / LONG_RANGE_CUTOFF
KEHALF = 7.199822675975274

ROWS = 16
ROW_W = 128
BLK = ROWS * ROW_W
NBLK = N_EDGES // BLK
NTILES = 32
BLK_PER_TILE = NBLK // NTILES
BLK_REM = NBLK - BLK_PER_TILE * NTILES
ACC_PAD = 100096
ACC_SLICE = ACC_PAD // 16

_MESH = plsc.VectorSubcoreMesh(
    core_axis_name="c", subcore_axis_name="s", num_cores=2, num_subcores=16
)


def _rsqrt_f32(s):
    xi = plsc.bitcast(s, jnp.int32)
    yi = jnp.int32(0x5F3759DF) - lax.shift_right_arithmetic(xi, 1)
    y = plsc.bitcast(yi, jnp.float32)
    half_s = 0.5 * s
    for _ in range(3):
        y = y * (1.5 - half_s * y * y)
    return y


def _edge_energy(d, qi, qj):
    s = d * d + 1.0
    rs = _rsqrt_f32(s)
    ds = s * rs
    e_shl = rs + ds * INV_LR2 - TWO_OVER_LR
    diff = (1.0 / d - rs) + (d - ds) * INV_LR2
    x = jnp.minimum(d * (1.0 / SHORT_RANGE_CUTOFF), 1.0)
    sw = x * x * x * (x * (x * 6.0 - 15.0) + 10.0)
    e = (KEHALF * qi * qj) * (e_shl + sw * diff)
    return jnp.where(d <= LONG_RANGE_CUTOFF, e, 0.0)


@functools.partial(
    pl.kernel,
    out_type=jax.ShapeDtypeStruct((2 * ACC_PAD,), jnp.float32),
    mesh=_MESH,
    compiler_params=pltpu.CompilerParams(needs_layout_passes=False),
    scratch_types=[
        pltpu.VMEM((N_NODES,), jnp.float32),
        pltpu.VMEM((ROWS, ROW_W), jnp.int32),
        pltpu.VMEM((ROWS, ROW_W), jnp.int32),
        pltpu.VMEM((ROWS, ROW_W), jnp.float32),
        pltpu.VMEM((ROWS, ROW_W), jnp.float32),
        pltpu.VMEM((ROWS, ROW_W), jnp.int32),
        pltpu.VMEM((ROWS, ROW_W), jnp.int32),
        pltpu.VMEM((ROWS, ROW_W), jnp.float32),
        pltpu.VMEM((ROWS, ROW_W), jnp.float32),
        pltpu.VMEM((ACC_SLICE,), jnp.float32),
        pltpu.VMEM_SHARED((ACC_PAD,), jnp.float32),
        pltpu.SemaphoreType.DMA,
        pltpu.SemaphoreType.DMA,
        pltpu.SemaphoreType.DMA,
        pltpu.SemaphoreType.DMA,
    ],
)
def _sc_energy(charges_hbm, dist_hbm, idxi_hbm, idxj_hbm, zeros_hbm, out_hbm,
               table_v, iiA, ijA, diA, eA, iiB, ijB, diB, eB, stage_v, acc_sh,
               semA_in, semB_in, semA_sc, semB_sc):
    c = lax.axis_index("c")
    s = lax.axis_index("s")
    wid = s * 2 + c

    setA = (iiA, ijA, diA, eA, semA_in, semA_sc)
    setB = (iiB, ijB, diB, eB, semB_in, semB_sc)

    def fire_in(bufs, bg):
        ii, ij, di, _, sem, _ = bufs
        return (
            pltpu.async_copy(idxi_hbm.at[bg], ii, sem),
            pltpu.async_copy(idxj_hbm.at[bg], ij, sem),
            pltpu.async_copy(dist_hbm.at[bg], di, sem),
        )

    def compute(bufs):
        ii_v, ij_v, di_v, e_v, _, _ = bufs

        def row_body(r, _):
            for j in range(ROW_W // 16):
                off = j * 16
                ii = ii_v[r, pl.ds(off, 16)]
                ij = ij_v[r, pl.ds(off, 16)]
                d = di_v[r, pl.ds(off, 16)]
                qi = plsc.load_gather(table_v, [ii])
                qj = plsc.load_gather(table_v, [ij])
                e_v[r, pl.ds(off, 16)] = _edge_energy(d, qi, qj)
            return 0

        lax.fori_loop(0, ROWS, row_body, 0)

    def fire_scatter(bufs):
        ii_v, _, _, e_v, _, sem = bufs
        return [
            pltpu.async_copy(e_v.at[r], acc_sh.at[ii_v.at[r]], sem, add=True)
            for r in range(ROWS)
        ]

    def drain(descs):
        for desc in descs:
            desc.wait()

    pltpu.sync_copy(charges_hbm, table_v)
    pltpu.sync_copy(zeros_hbm.at[pl.ds(s * ACC_SLICE, ACC_SLICE)], stage_v)
    pltpu.sync_copy(stage_v, acc_sh.at[pl.ds(s * ACC_SLICE, ACC_SLICE)])
    plsc.subcore_barrier()

    start_blk = wid * BLK_PER_TILE + jnp.minimum(wid, BLK_REM)

    def pair_body(p, _):
        inA = fire_in(setA, start_blk + 2 * p)
        inB = fire_in(setB, start_blk + 2 * p + 1)
        drain(inA)
        compute(setA)
        scA = fire_scatter(setA)
        drain(inB)
        compute(setB)
        drain(scA)
        scB = fire_scatter(setB)
        drain(scB)
        return 0

    lax.fori_loop(0, BLK_PER_TILE // 2, pair_body, 0)

    def tail_block(bg):
        inB = fire_in(setB, bg)
        drain(inB)
        compute(setB)
        drain(fire_scatter(setB))

    tail_block(start_blk + BLK_PER_TILE - 1)

    @pl.when(wid < BLK_REM)
    def _():
        tail_block(start_blk + BLK_PER_TILE)

    plsc.subcore_barrier()
    pltpu.sync_copy(acc_sh.at[pl.ds(s * ACC_SLICE, ACC_SLICE)], stage_v)
    pltpu.sync_copy(
        stage_v, out_hbm.at[pl.ds(c * ACC_PAD + s * ACC_SLICE, ACC_SLICE)]
    )


def _combine_body(p_ref, o_ref):
    o_ref[...] = p_ref[0] + p_ref[1]


def kernel(atomic_charges, distances, idx_i, idx_j):
    idx_i = idx_i.astype(jnp.int32).reshape(NBLK, ROWS, ROW_W)
    idx_j = idx_j.astype(jnp.int32).reshape(NBLK, ROWS, ROW_W)
    dist = distances.reshape(NBLK, ROWS, ROW_W)
    zeros = jnp.zeros((ACC_PAD,), jnp.float32)
    part = _sc_energy(atomic_charges, dist, idx_i, idx_j, zeros)
    part = part.reshape(2, ACC_PAD // 128, 128)
    summed = pl.pallas_call(
        _combine_body,
        out_shape=jax.ShapeDtypeStruct((ACC_PAD // 128, 128), jnp.float32),
    )(part)
    return summed.reshape(ACC_PAD)[:N_NODES]
